# baseline (speedup 1.0000x reference)
import functools

import jax
import jax.numpy as jnp
from jax import lax
from jax.experimental import pallas as pl
from jax.experimental.pallas import tpu as pltpu
from jax.experimental.pallas import tpu_sc as plsc

_T, _VOCAB, _DIM, _B = 26, 100000, 128, 4096
_N = _T * _B
_NW = 32
_PW = _N // _NW
_CH = 128
_NCH = _PW // _CH
_WIN = 4


def _sc_gather(indices, qflat, sflat, bflat):
  mesh = plsc.VectorSubcoreMesh(core_axis_name="c", subcore_axis_name="s")

  @functools.partial(
      pl.kernel,
      mesh=mesh,
      compiler_params=pltpu.CompilerParams(use_tc_tiling_on_sc=False),
      out_type=[
          jax.ShapeDtypeStruct((_N, _DIM // 4), jnp.int32),
          jax.ShapeDtypeStruct((_N,), jnp.float32),
          jax.ShapeDtypeStruct((_N,), jnp.float32),
      ],
      scratch_types=[
          pltpu.VMEM((_PW,), jnp.int32),
          pltpu.VMEM((_NCH, _CH), jnp.int32),
          pltpu.VMEM((_PW, _DIM // 4), jnp.int32),
          pltpu.VMEM((_PW,), jnp.float32),
          pltpu.VMEM((_PW,), jnp.float32),
          pltpu.SemaphoreType.DMA,
      ],
  )
  def k(idx_hbm, q_hbm, s_hbm, b_hbm, qg_hbm, sg_hbm, bg_hbm,
        idx_raw, idx_v, rows_v, s_v, b_v, sem):
    wid = lax.axis_index("s") * 2 + lax.axis_index("c")
    base = wid * _PW

    pltpu.sync_copy(idx_hbm.at[pl.ds(base, _PW)], idx_raw)

    for c in range(_NCH):
      t_c = lax.shift_right_logical(base + c * _CH, 12)
      off = t_c * _VOCAB
      for j in range(_CH // 16):
        v = idx_raw[pl.ds(c * _CH + j * 16, 16)]
        idx_v[c, pl.ds(j * 16, 16)] = v + off

    pending = []
    for c in range(_NCH):
      sl = pl.ds(c * _CH, _CH)
      pending.append(pltpu.async_copy(q_hbm.at[idx_v.at[c]], rows_v.at[sl], sem))
      pending.append(pltpu.async_copy(s_hbm.at[idx_v.at[c]], s_v.at[sl], sem))
      pending.append(pltpu.async_copy(b_hbm.at[idx_v.at[c]], b_v.at[sl], sem))
      while len(pending) > 3 * _WIN:
        pending.pop(0).wait()
    while pending:
      pending.pop(0).wait()

    pltpu.sync_copy(rows_v, qg_hbm.at[pl.ds(base, _PW)])
    pltpu.sync_copy(s_v, sg_hbm.at[pl.ds(base, _PW)])
    pltpu.sync_copy(b_v, bg_hbm.at[pl.ds(base, _PW)])

  return k(indices, qflat, sflat, bflat)


def _tc_dequant(qg, sg, bg):
  bb = 1024

  def body(q_ref, s_ref, b_ref, o_ref):
    q = q_ref[0].astype(jnp.float32)
    x = q * s_ref[0] + b_ref[0]
    bits = lax.bitcast_convert_type(x, jnp.int32)
    sgn = lax.shift_right_logical(bits, 16) & 0x8000
    mag = bits & 0x7FFFFFFF
    rnd = mag + 0xFFF + (lax.shift_right_logical(mag, 13) & 1)
    h = lax.shift_right_logical(rnd, 13) - 0x1C000
    h = jnp.where(mag < 0x38800000, 0, h)
    o_ref[...] = (sgn | h).astype(jnp.uint16)

  return pl.pallas_call(
      body,
      grid=(_T, _B // bb),
      in_specs=[
          pl.BlockSpec((1, bb, _DIM), lambda t, b: (t, b, 0)),
          pl.BlockSpec((1, bb, 1), lambda t, b: (t, b, 0)),
          pl.BlockSpec((1, bb, 1), lambda t, b: (t, b, 0)),
      ],
      out_specs=pl.BlockSpec((bb, _DIM), lambda t, b: (b, t)),
      out_shape=jax.ShapeDtypeStruct((_B, _T * _DIM), jnp.uint16),
  )(qg, sg, bg)


def kernel(indices, offsets, qweights, scales, biases):
  del offsets
  qflat = lax.bitcast_convert_type(
      qweights.reshape(_T * _VOCAB, _DIM // 4, 4), jnp.int32)
  sflat = scales.reshape(_T * _VOCAB)
  bflat = biases.reshape(_T * _VOCAB)
  qg, sg, bg = _sc_gather(indices, qflat, sflat, bflat)
  qg_u8 = lax.bitcast_convert_type(qg, jnp.uint8).reshape(_N, _DIM)
  out_u16 = _tc_dequant(
      qg_u8.reshape(_T, _B, _DIM),
      sg.reshape(_T, _B, 1),
      bg.reshape(_T, _B, 1),
  )
  return lax.bitcast_convert_type(out_u16, jnp.float16)


# speedup vs baseline: 16.4537x; 16.4537x over previous
import functools

import jax
import jax.numpy as jnp
from jax import lax
from jax.experimental import pallas as pl
from jax.experimental.pallas import tpu as pltpu
from jax.experimental.pallas import tpu_sc as plsc

_T, _VOCAB, _DIM, _B = 26, 100000, 128, 4096
_N = _T * _B
_NW = 32
_PW = _N // _NW
_CH = 128
_NCH = _PW // _CH
_NB = 4
_G = _T * _VOCAB // 4


def _sc_gather(indices, qpack, sflat, bflat):
  mesh = plsc.VectorSubcoreMesh(core_axis_name="c", subcore_axis_name="s")

  @functools.partial(
      pl.kernel,
      mesh=mesh,
      compiler_params=pltpu.CompilerParams(use_tc_tiling_on_sc=False),
      out_type=[
          jax.ShapeDtypeStruct((_N, _DIM), jnp.int32),
          jax.ShapeDtypeStruct((_N,), jnp.float32),
          jax.ShapeDtypeStruct((_N,), jnp.float32),
      ],
      scratch_types=[
          pltpu.VMEM((_PW,), jnp.int32),
          pltpu.VMEM((_NCH, _CH), jnp.int32),
          pltpu.VMEM((_NCH, _CH), jnp.int32),
          pltpu.VMEM((_NB, _CH, _DIM), jnp.int32),
          pltpu.VMEM((_PW,), jnp.float32),
          pltpu.VMEM((_PW,), jnp.float32),
          pltpu.SemaphoreType.DMA,
          pltpu.SemaphoreType.DMA,
          pltpu.SemaphoreType.DMA,
      ],
  )
  def k(idx_hbm, q_hbm, s_hbm, b_hbm, qg_hbm, sg_hbm, bg_hbm,
        idx_raw, idx_v, idx_g, gran_v, s_v, b_v, sem_g, sem_o, sem_sb):
    wid = lax.axis_index("s") * 2 + lax.axis_index("c")
    base = wid * _PW

    pltpu.sync_copy(idx_hbm.at[pl.ds(base, _PW)], idx_raw)

    for c in range(_NCH):
      t_c = lax.shift_right_logical(base + c * _CH, 12)
      off = t_c * _VOCAB
      for j in range(_CH // 16):
        v = idx_raw[pl.ds(c * _CH + j * 16, 16)] + off
        idx_v[c, pl.ds(j * 16, 16)] = v
        idx_g[c, pl.ds(j * 16, 16)] = lax.shift_right_logical(v, 2)

    sb_pending = []
    for c in range(_NCH):
      sl = pl.ds(c * _CH, _CH)
      rows = idx_v.at[c]
      sb_pending.append(pltpu.async_copy(s_hbm.at[rows], s_v.at[sl], sem_sb))
      sb_pending.append(pltpu.async_copy(b_hbm.at[rows], b_v.at[sl], sem_sb))
      while len(sb_pending) > 8:
        sb_pending.pop(0).wait()

    g_h = [None] * _NB
    o_h = [None] * _NB
    for c in range(_NCH):
      b = c % _NB
      if o_h[b] is not None:
        o_h[b].wait()
      g_h[b] = pltpu.async_copy(q_hbm.at[idx_g.at[c]], gran_v.at[b], sem_g)
      if c >= _NB - 1:
        cd = c - (_NB - 1)
        bd = cd % _NB
        g_h[bd].wait()
        o_h[bd] = pltpu.async_copy(
            gran_v.at[bd], qg_hbm.at[pl.ds(base + cd * _CH, _CH)], sem_o)
    for cd in range(_NCH - _NB + 1, _NCH):
      bd = cd % _NB
      g_h[bd].wait()
      o_h[bd] = pltpu.async_copy(
          gran_v.at[bd], qg_hbm.at[pl.ds(base + cd * _CH, _CH)], sem_o)
    for h in o_h:
      if h is not None:
        h.wait()
    while sb_pending:
      sb_pending.pop(0).wait()

    pltpu.sync_copy(s_v, sg_hbm.at[pl.ds(base, _PW)])
    pltpu.sync_copy(b_v, bg_hbm.at[pl.ds(base, _PW)])

  return k(indices, qpack, sflat, bflat)


def _tc_dequant(qg, kshift, sg, bg):
  bb = 1024

  def body(q_ref, k_ref, s_ref, b_ref, o_ref):
    q = lax.shift_right_logical(q_ref[0], k_ref[0]) & 0xFF
    x = q.astype(jnp.float32) * s_ref[0] + b_ref[0]
    bits = lax.bitcast_convert_type(x, jnp.int32)
    sgn = lax.shift_right_logical(bits, 16) & 0x8000
    mag = bits & 0x7FFFFFFF
    rnd = mag + 0xFFF + (lax.shift_right_logical(mag, 13) & 1)
    h = lax.shift_right_logical(rnd, 13) - 0x1C000
    h = jnp.where(mag < 0x38800000, 0, h)
    o_ref[...] = (sgn | h).astype(jnp.uint16)

  return pl.pallas_call(
      body,
      grid=(_T, _B // bb),
      in_specs=[
          pl.BlockSpec((1, bb, _DIM), lambda t, b: (t, b, 0)),
          pl.BlockSpec((1, bb, 1), lambda t, b: (t, b, 0)),
          pl.BlockSpec((1, bb, 1), lambda t, b: (t, b, 0)),
          pl.BlockSpec((1, bb, 1), lambda t, b: (t, b, 0)),
      ],
      out_specs=pl.BlockSpec((bb, _DIM), lambda t, b: (b, t)),
      out_shape=jax.ShapeDtypeStruct((_B, _T * _DIM), jnp.uint16),
  )(qg, kshift, sg, bg)


def kernel(indices, offsets, qweights, scales, biases):
  del offsets
  q4 = qweights.reshape(_G, 4, _DIM)
  qj = [q4[:, j, :].astype(jnp.uint32) for j in range(4)]
  qpack = lax.bitcast_convert_type(
      qj[0] | (qj[1] << 8) | (qj[2] << 16) | (qj[3] << 24), jnp.int32)
  sflat = scales.reshape(_T * _VOCAB)
  bflat = biases.reshape(_T * _VOCAB)
  qg, sg, bg = _sc_gather(indices, qpack, sflat, bflat)
  kshift = ((indices & 3) * 8).reshape(_T, _B, 1)
  out_u16 = _tc_dequant(
      qg.reshape(_T, _B, _DIM),
      kshift,
      sg.reshape(_T, _B, 1),
      bg.reshape(_T, _B, 1),
  )
  return lax.bitcast_convert_type(out_u16, jnp.float16)


# speedup vs baseline: 31.9085x; 1.9393x over previous
import functools

import jax
import jax.numpy as jnp
from jax import lax
from jax.experimental import pallas as pl
from jax.experimental.pallas import tpu as pltpu
from jax.experimental.pallas import tpu_sc as plsc

_T, _VOCAB, _DIM, _B = 26, 100000, 128, 4096
_N = _T * _B
_NW = 32
_PW = _N // _NW
_CH = 128
_NCH = _PW // _CH
_NB = 4
_G = _T * _VOCAB // 4


def _sc_gather(indices, qpack, sflat, bflat):
  mesh = plsc.VectorSubcoreMesh(core_axis_name="c", subcore_axis_name="s")

  @functools.partial(
      pl.kernel,
      mesh=mesh,
      compiler_params=pltpu.CompilerParams(use_tc_tiling_on_sc=False),
      out_type=[
          jax.ShapeDtypeStruct((_N, _DIM), jnp.int32),
          jax.ShapeDtypeStruct((_N,), jnp.float32),
          jax.ShapeDtypeStruct((_N,), jnp.float32),
      ],
      scratch_types=[
          pltpu.VMEM((_PW,), jnp.int32),
          pltpu.VMEM((_NCH, _CH), jnp.int32),
          pltpu.VMEM((_NCH, _CH), jnp.int32),
          pltpu.VMEM((_NB, _CH, _DIM), jnp.int32),
          pltpu.VMEM((_PW,), jnp.float32),
          pltpu.VMEM((_PW,), jnp.float32),
          pltpu.SemaphoreType.DMA,
          pltpu.SemaphoreType.DMA,
          pltpu.SemaphoreType.DMA,
      ],
  )
  def k(idx_hbm, q_hbm, s_hbm, b_hbm, qg_hbm, sg_hbm, bg_hbm,
        idx_raw, idx_v, idx_g, gran_v, s_v, b_v, sem_g, sem_o, sem_sb):
    wid = lax.axis_index("s") * 2 + lax.axis_index("c")
    base = wid * _PW

    pltpu.sync_copy(idx_hbm.at[pl.ds(base, _PW)], idx_raw)

    for c in range(_NCH):
      t_c = lax.shift_right_logical(base + c * _CH, 12)
      off = t_c * _VOCAB
      for j in range(_CH // 16):
        v = idx_raw[pl.ds(c * _CH + j * 16, 16)] + off
        idx_v[c, pl.ds(j * 16, 16)] = v
        idx_g[c, pl.ds(j * 16, 16)] = lax.shift_right_logical(v, 2)

    sb_pending = []
    for c in range(_NCH):
      sl = pl.ds(c * _CH, _CH)
      rows = idx_v.at[c]
      sb_pending.append(pltpu.async_copy(s_hbm.at[rows], s_v.at[sl], sem_sb))
      sb_pending.append(pltpu.async_copy(b_hbm.at[rows], b_v.at[sl], sem_sb))
      while len(sb_pending) > 8:
        sb_pending.pop(0).wait()

    g_h = [None] * _NB
    o_h = [None] * _NB
    for c in range(_NCH):
      b = c % _NB
      if o_h[b] is not None:
        o_h[b].wait()
      g_h[b] = pltpu.async_copy(q_hbm.at[idx_g.at[c]], gran_v.at[b], sem_g)
      if c >= _NB - 1:
        cd = c - (_NB - 1)
        bd = cd % _NB
        g_h[bd].wait()
        o_h[bd] = pltpu.async_copy(
            gran_v.at[bd], qg_hbm.at[pl.ds(base + cd * _CH, _CH)], sem_o)
    for cd in range(_NCH - _NB + 1, _NCH):
      bd = cd % _NB
      g_h[bd].wait()
      o_h[bd] = pltpu.async_copy(
          gran_v.at[bd], qg_hbm.at[pl.ds(base + cd * _CH, _CH)], sem_o)
    for h in o_h:
      if h is not None:
        h.wait()
    while sb_pending:
      sb_pending.pop(0).wait()

    pltpu.sync_copy(s_v, sg_hbm.at[pl.ds(base, _PW)])
    pltpu.sync_copy(b_v, bg_hbm.at[pl.ds(base, _PW)])

  return k(indices, qpack, sflat, bflat)


def _tc_repack(q4):
  rr = 2600

  def body(q_ref, o_ref):
    q = q_ref[...]
    w = (q[:, 0, :].astype(jnp.uint32)
         | (q[:, 1, :].astype(jnp.uint32) << 8)
         | (q[:, 2, :].astype(jnp.uint32) << 16)
         | (q[:, 3, :].astype(jnp.uint32) << 24))
    o_ref[...] = lax.bitcast_convert_type(w, jnp.int32)

  return pl.pallas_call(
      body,
      grid=(_G // rr,),
      in_specs=[pl.BlockSpec((rr, 4, _DIM), lambda g: (g, 0, 0))],
      out_specs=pl.BlockSpec((rr, _DIM), lambda g: (g, 0)),
      out_shape=jax.ShapeDtypeStruct((_G, _DIM), jnp.int32),
  )(q4)


def _tc_dequant(qg, kshift, sg, bg):
  bb = 1024

  def body(q_ref, k_ref, s_ref, b_ref, o_ref):
    q = lax.shift_right_logical(q_ref[0], k_ref[0]) & 0xFF
    x = q.astype(jnp.float32) * s_ref[0] + b_ref[0]
    bits = lax.bitcast_convert_type(x, jnp.int32)
    sgn = lax.shift_right_logical(bits, 16) & 0x8000
    mag = bits & 0x7FFFFFFF
    rnd = mag + 0xFFF + (lax.shift_right_logical(mag, 13) & 1)
    h = lax.shift_right_logical(rnd, 13) - 0x1C000
    h = jnp.where(mag < 0x38800000, 0, h)
    o_ref[...] = (sgn | h).astype(jnp.uint16)

  return pl.pallas_call(
      body,
      grid=(_T, _B // bb),
      in_specs=[
          pl.BlockSpec((1, bb, _DIM), lambda t, b: (t, b, 0)),
          pl.BlockSpec((1, bb, 1), lambda t, b: (t, b, 0)),
          pl.BlockSpec((1, bb, 1), lambda t, b: (t, b, 0)),
          pl.BlockSpec((1, bb, 1), lambda t, b: (t, b, 0)),
      ],
      out_specs=pl.BlockSpec((bb, _DIM), lambda t, b: (b, t)),
      out_shape=jax.ShapeDtypeStruct((_B, _T * _DIM), jnp.uint16),
  )(qg, kshift, sg, bg)


def kernel(indices, offsets, qweights, scales, biases):
  del offsets
  qpack = _tc_repack(qweights.reshape(_G, 4, _DIM))
  sflat = scales.reshape(_T * _VOCAB)
  bflat = biases.reshape(_T * _VOCAB)
  qg, sg, bg = _sc_gather(indices, qpack, sflat, bflat)
  kshift = ((indices & 3) * 8).reshape(_T, _B, 1)
  out_u16 = _tc_dequant(
      qg.reshape(_T, _B, _DIM),
      kshift,
      sg.reshape(_T, _B, 1),
      bg.reshape(_T, _B, 1),
  )
  return lax.bitcast_convert_type(out_u16, jnp.float16)


# speedup vs baseline: 72.8357x; 2.2826x over previous
import functools

import jax
import jax.numpy as jnp
from jax import lax
from jax.experimental import pallas as pl
from jax.experimental.pallas import tpu as pltpu
from jax.experimental.pallas import tpu_sc as plsc

_T, _VOCAB, _DIM, _B = 26, 100000, 128, 4096
_N = _T * _B
_NW = 32
_PW = _N // _NW
_CH = 128
_NCH = _PW // _CH
_NB = 4
_G = _T * _VOCAB // 4


def _sc_gather(indices, qpack, sflat, bflat):
  mesh = plsc.VectorSubcoreMesh(core_axis_name="c", subcore_axis_name="s")

  @functools.partial(
      pl.kernel,
      mesh=mesh,
      compiler_params=pltpu.CompilerParams(use_tc_tiling_on_sc=False),
      out_type=[
          jax.ShapeDtypeStruct((_N, _DIM), jnp.int32),
          jax.ShapeDtypeStruct((_N,), jnp.float32),
          jax.ShapeDtypeStruct((_N,), jnp.float32),
      ],
      scratch_types=[
          pltpu.VMEM((_PW,), jnp.int32),
          pltpu.VMEM((_NCH, _CH), jnp.int32),
          pltpu.VMEM((_NCH, _CH), jnp.int32),
          pltpu.VMEM((_NB, _CH, _DIM), jnp.int32),
          pltpu.VMEM((_PW,), jnp.float32),
          pltpu.VMEM((_PW,), jnp.float32),
          pltpu.SemaphoreType.DMA,
          pltpu.SemaphoreType.DMA,
          pltpu.SemaphoreType.DMA,
      ],
  )
  def k(idx_hbm, q_hbm, s_hbm, b_hbm, qg_hbm, sg_hbm, bg_hbm,
        idx_raw, idx_v, idx_g, gran_v, s_v, b_v, sem_g, sem_o, sem_sb):
    wid = lax.axis_index("s") * 2 + lax.axis_index("c")
    base = wid * _PW

    pltpu.sync_copy(idx_hbm.at[pl.ds(base, _PW)], idx_raw)

    for c in range(_NCH):
      t_c = lax.shift_right_logical(base + c * _CH, 12)
      off = t_c * _VOCAB
      for j in range(_CH // 16):
        v = idx_raw[pl.ds(c * _CH + j * 16, 16)] + off
        idx_v[c, pl.ds(j * 16, 16)] = v
        idx_g[c, pl.ds(j * 16, 16)] = lax.shift_right_logical(v, 2)

    sb_pending = []
    for c in range(_NCH):
      sl = pl.ds(c * _CH, _CH)
      rows = idx_v.at[c]
      sb_pending.append(pltpu.async_copy(s_hbm.at[rows], s_v.at[sl], sem_sb))
      sb_pending.append(pltpu.async_copy(b_hbm.at[rows], b_v.at[sl], sem_sb))
      while len(sb_pending) > 8:
        sb_pending.pop(0).wait()

    g_h = [None] * _NB
    o_h = [None] * _NB
    for c in range(_NCH):
      b = c % _NB
      if o_h[b] is not None:
        o_h[b].wait()
      g_h[b] = pltpu.async_copy(q_hbm.at[idx_g.at[c]], gran_v.at[b], sem_g)
      if c >= _NB - 1:
        cd = c - (_NB - 1)
        bd = cd % _NB
        g_h[bd].wait()
        o_h[bd] = pltpu.async_copy(
            gran_v.at[bd], qg_hbm.at[pl.ds(base + cd * _CH, _CH)], sem_o)
    for cd in range(_NCH - _NB + 1, _NCH):
      bd = cd % _NB
      g_h[bd].wait()
      o_h[bd] = pltpu.async_copy(
          gran_v.at[bd], qg_hbm.at[pl.ds(base + cd * _CH, _CH)], sem_o)
    for h in o_h:
      if h is not None:
        h.wait()
    while sb_pending:
      sb_pending.pop(0).wait()

    pltpu.sync_copy(s_v, sg_hbm.at[pl.ds(base, _PW)])
    pltpu.sync_copy(b_v, bg_hbm.at[pl.ds(base, _PW)])

  return k(indices, qpack, sflat, bflat)


def _tc_repack(qflat):
  rr = 2600

  def body(q_ref, o_ref):
    o_ref[...] = pltpu.bitcast(q_ref[...], jnp.int32)

  return pl.pallas_call(
      body,
      grid=(_G // rr,),
      in_specs=[pl.BlockSpec((4 * rr, _DIM), lambda g: (g, 0))],
      out_specs=pl.BlockSpec((rr, _DIM), lambda g: (g, 0)),
      out_shape=jax.ShapeDtypeStruct((_G, _DIM), jnp.int32),
  )(qflat)


def _tc_dequant(qg, kshift, sg, bg):
  bb = 1024

  def body(q_ref, k_ref, s_ref, b_ref, o_ref):
    q = lax.shift_right_logical(q_ref[0], k_ref[0]) & 0xFF
    x = q.astype(jnp.float32) * s_ref[0] + b_ref[0]
    bits = lax.bitcast_convert_type(x, jnp.int32)
    sgn = lax.shift_right_logical(bits, 16) & 0x8000
    mag = bits & 0x7FFFFFFF
    rnd = mag + 0xFFF + (lax.shift_right_logical(mag, 13) & 1)
    h = lax.shift_right_logical(rnd, 13) - 0x1C000
    h = jnp.where(mag < 0x38800000, 0, h)
    o_ref[...] = (sgn | h).astype(jnp.uint16)

  return pl.pallas_call(
      body,
      grid=(_T, _B // bb),
      in_specs=[
          pl.BlockSpec((1, bb, _DIM), lambda t, b: (t, b, 0)),
          pl.BlockSpec((1, bb, 1), lambda t, b: (t, b, 0)),
          pl.BlockSpec((1, bb, 1), lambda t, b: (t, b, 0)),
          pl.BlockSpec((1, bb, 1), lambda t, b: (t, b, 0)),
      ],
      out_specs=pl.BlockSpec((bb, _DIM), lambda t, b: (b, t)),
      out_shape=jax.ShapeDtypeStruct((_B, _T * _DIM), jnp.uint16),
  )(qg, kshift, sg, bg)


def kernel(indices, offsets, qweights, scales, biases):
  del offsets
  qpack = _tc_repack(qweights.reshape(_T * _VOCAB, _DIM))
  sflat = scales.reshape(_T * _VOCAB)
  bflat = biases.reshape(_T * _VOCAB)
  qg, sg, bg = _sc_gather(indices, qpack, sflat, bflat)
  kshift = ((indices & 3) * 8).reshape(_T, _B, 1)
  out_u16 = _tc_dequant(
      qg.reshape(_T, _B, _DIM),
      kshift,
      sg.reshape(_T, _B, 1),
      bg.reshape(_T, _B, 1),
  )
  return lax.bitcast_convert_type(out_u16, jnp.float16)
